# baseline (speedup 1.0000x reference)
import functools

import jax
import jax.numpy as jnp
from jax import lax
from jax.experimental import pallas as pl
from jax.experimental.pallas import tpu as pltpu
from jax.experimental.pallas import tpu_sc as plsc

T, REPR, FEAT, HID, GATE, E, MOEH, ACT = 2048, 2048, 1024, 1024, 256, 16, 512, 12
BT = 128
NTT = T // BT
BE = 128
NT = 48
NPAD = NT * BE
NA = 2 * T

_INTERPRET = False

_NC, _NS = 2, 16
_NW = _NC * _NS
_CHUNK = 64


def _stage_a_body(obs_ref, Wt_ref, bt_ref, lng_ref, lnb_ref, Wp1_ref, bp1_ref,
                  Wg1_ref, bg1_ref, Wg2_ref, bg2_ref,
                  x_ref, tv0_ref, tv1_ref, p_ref, et_ref, aux_ref,
                  carry0, carry1, sp_acc, ls_acc, i0_s, i1_s, r0_s, r1_s):
    i = pl.program_id(0)
    h = jnp.dot(obs_ref[...], Wt_ref[...], preferred_element_type=jnp.float32) + bt_ref[...]
    mu_ = jnp.mean(h, axis=-1, keepdims=True)
    var = jnp.mean((h - mu_) ** 2, axis=-1, keepdims=True)
    h = (h - mu_) * jax.lax.rsqrt(var + 1e-5) * lng_ref[...] + lnb_ref[...]
    h = jnp.tanh(h)

    x = jnp.maximum(jnp.dot(h, Wp1_ref[...], preferred_element_type=jnp.float32) + bp1_ref[...], 0.0)
    x_ref[...] = x

    gh = jnp.maximum(jnp.dot(x, Wg1_ref[...], preferred_element_type=jnp.float32) + bg1_ref[...], 0.0)
    logits = jnp.dot(gh, Wg2_ref[...], preferred_element_type=jnp.float32) + bg2_ref[...]

    m = jnp.max(logits, axis=-1, keepdims=True)
    ex = jnp.exp(logits - m)
    probs = ex / jnp.sum(ex, axis=-1, keepdims=True)

    iot = jax.lax.broadcasted_iota(jnp.int32, probs.shape, 1)
    m0 = jnp.max(probs, axis=-1, keepdims=True)
    i0 = jnp.argmax(probs, axis=-1).reshape(BT, 1)
    masked = jnp.where(iot == i0, -1.0, probs)
    m1 = jnp.max(masked, axis=-1, keepdims=True)
    i1 = jnp.argmax(masked, axis=-1).reshape(BT, 1)
    s = m0 + m1 + 1e-9
    tv0 = m0 / s
    tv1 = m1 / s
    tv0_ref[...] = tv0
    tv1_ref[...] = tv1

    g0 = (iot == i0).astype(jnp.float32)
    g1 = (iot == i1).astype(jnp.float32)
    ir = jax.lax.broadcasted_iota(jnp.int32, (BT, BT), 0)
    ic = jax.lax.broadcasted_iota(jnp.int32, (BT, BT), 1)
    st = (ic < ir).astype(jnp.float32)

    @pl.when(i == 0)
    def _():
        carry0[...] = jnp.zeros_like(carry0)
        carry1[...] = jnp.zeros_like(carry1)
        sp_acc[...] = jnp.zeros_like(sp_acc)
        ls_acc[...] = jnp.zeros_like(ls_acc)

    cum0 = jnp.dot(st, g0, preferred_element_type=jnp.float32) + carry0[...]
    cum1 = jnp.dot(st, g1, preferred_element_type=jnp.float32) + carry1[...]
    rows = pl.ds(i * BT, BT)
    i0_s[rows, :] = i0
    i1_s[rows, :] = i1
    r0_s[rows, :] = jnp.sum(g0 * cum0, axis=1, keepdims=True)
    r1_s[rows, :] = jnp.sum(g1 * cum1, axis=1, keepdims=True)
    carry0[...] += jnp.sum(g0, axis=0, keepdims=True)
    carry1[...] += jnp.sum(g1, axis=0, keepdims=True)

    sp_acc[...] += jnp.sum(probs, axis=0, keepdims=True)
    ls_acc[...] += jnp.sum(g0 * (tv0 > 0.0).astype(jnp.float32)
                           + g1 * (tv1 > 0.0).astype(jnp.float32), axis=0, keepdims=True)

    @pl.when(i == NTT - 1)
    def _():
        c0 = carry0[...]
        c1 = carry1[...]
        pc = jnp.ceil((c0 + c1) / BE) * BE
        fr = jax.lax.broadcasted_iota(jnp.int32, (E, E), 0)
        fc = jax.lax.broadcasted_iota(jnp.int32, (E, E), 1)
        mstrict = (fr < fc).astype(jnp.float32)
        offp = jnp.dot(pc, mstrict, preferred_element_type=jnp.float32)
        cum_end = offp + pc

        iota_e = jax.lax.broadcasted_iota(jnp.int32, (T, E), 1)
        oh0 = (iota_e == i0_s[...]).astype(jnp.float32)
        oh1 = (iota_e == i1_s[...]).astype(jnp.float32)
        p0 = jnp.sum(oh0 * offp, axis=1, keepdims=True) + r0_s[...]
        p1 = jnp.sum(oh1 * (offp + c0), axis=1, keepdims=True) + r1_s[...]
        p_ref[0:T, :] = p0.astype(jnp.int32)
        p_ref[T:NA, :] = p1.astype(jnp.int32)

        it = (jax.lax.broadcasted_iota(jnp.int32, (NT, E), 0) * BE).astype(jnp.float32)
        et = jnp.sum((cum_end <= it).astype(jnp.float32), axis=1, keepdims=True)
        et_ref[...] = et.astype(jnp.int32)

        aux_ref[...] = ((E / (T * float(T))) * jnp.sum(sp_acc[...] * ls_acc[...])).reshape(1, 1)


def _stage_a(obs, Wt, bt, ln_g, ln_b, Wp1, bp1, Wg1, bg1, Wg2, bg2):
    row = lambda v: v.reshape(1, -1)
    const = lambda i: (0, 0)
    tok = lambda i: (i, 0)
    return pl.pallas_call(
        _stage_a_body,
        grid=(NTT,),
        in_specs=[
            pl.BlockSpec((BT, REPR), tok),
            pl.BlockSpec((REPR, FEAT), const),
            pl.BlockSpec((1, FEAT), const),
            pl.BlockSpec((1, FEAT), const),
            pl.BlockSpec((1, FEAT), const),
            pl.BlockSpec((FEAT, HID), const),
            pl.BlockSpec((1, HID), const),
            pl.BlockSpec((HID, GATE), const),
            pl.BlockSpec((1, GATE), const),
            pl.BlockSpec((GATE, E), const),
            pl.BlockSpec((1, E), const),
        ],
        out_specs=[
            pl.BlockSpec((BT, HID), tok),
            pl.BlockSpec((BT, 1), tok),
            pl.BlockSpec((BT, 1), tok),
            pl.BlockSpec((NA, 1), const),
            pl.BlockSpec((NT, 1), const),
            pl.BlockSpec((1, 1), const),
        ],
        out_shape=[
            jax.ShapeDtypeStruct((T, HID), jnp.float32),
            jax.ShapeDtypeStruct((T, 1), jnp.float32),
            jax.ShapeDtypeStruct((T, 1), jnp.float32),
            jax.ShapeDtypeStruct((NA, 1), jnp.int32),
            jax.ShapeDtypeStruct((NT, 1), jnp.int32),
            jax.ShapeDtypeStruct((1, 1), jnp.float32),
        ],
        scratch_shapes=[pltpu.VMEM((1, E), jnp.float32)] * 4
        + [pltpu.VMEM((T, 1), jnp.int32)] * 2
        + [pltpu.VMEM((T, 1), jnp.float32)] * 2,
        interpret=_INTERPRET,
    )(obs, Wt, row(bt), row(ln_g), row(ln_b), Wp1, row(bp1), Wg1, row(bg1), Wg2, row(bg2))


def _sc_dispatch(x, p):
    @functools.partial(
        pl.kernel,
        mesh=plsc.VectorSubcoreMesh(core_axis_name="c", subcore_axis_name="s"),
        out_type=jax.ShapeDtypeStruct((NPAD, HID), jnp.float32),
        scratch_types=[
            pltpu.VMEM((_CHUNK,), jnp.int32),
            pltpu.VMEM((_CHUNK, HID), jnp.float32),
            pltpu.SemaphoreType.DMA,
        ],
    )
    def k(x_hbm, p_hbm, xs_hbm, idx_v, rows_v, sem):
        wid = lax.axis_index("s") * _NC + lax.axis_index("c")
        per_w = NA // _NW
        for c in range(per_w // _CHUNK):
            base = wid * per_w + c * _CHUNK
            tbase = jnp.where(base >= T, base - T, base)
            pltpu.sync_copy(p_hbm.at[pl.ds(base, _CHUNK)], idx_v)
            pltpu.sync_copy(x_hbm.at[pl.ds(tbase, _CHUNK)], rows_v)
            pltpu.async_copy(rows_v, xs_hbm.at[idx_v], sem).wait()

    return k(x, p)


def _sc_combine(ys, p):
    @functools.partial(
        pl.kernel,
        mesh=plsc.VectorSubcoreMesh(core_axis_name="c", subcore_axis_name="s"),
        out_type=jax.ShapeDtypeStruct((NA, HID), jnp.float32),
        scratch_types=[
            pltpu.VMEM((_CHUNK,), jnp.int32),
            pltpu.VMEM((_CHUNK, HID), jnp.float32),
            pltpu.SemaphoreType.DMA,
        ],
    )
    def k(ys_hbm, p_hbm, gy_hbm, idx_v, rows_v, sem):
        wid = lax.axis_index("s") * _NC + lax.axis_index("c")
        per_w = NA // _NW
        for c in range(per_w // _CHUNK):
            base = wid * per_w + c * _CHUNK
            pltpu.sync_copy(p_hbm.at[pl.ds(base, _CHUNK)], idx_v)
            pltpu.async_copy(ys_hbm.at[idx_v], rows_v, sem).wait()
            pltpu.sync_copy(rows_v, gy_hbm.at[pl.ds(base, _CHUNK)])

    return k(ys, p)


def _stage_c_body(et_sref, xs_ref, We1_ref, be1_ref, We2_ref, be2_ref, ys_ref):
    etv = et_sref[pl.program_id(0)]

    @pl.when(etv < E)
    def _():
        xb = xs_ref[...].astype(jnp.bfloat16)
        eh = jnp.maximum(jnp.dot(xb, We1_ref[0], preferred_element_type=jnp.float32)
                         + be1_ref[0], 0.0)
        ys_ref[...] = jnp.dot(eh.astype(jnp.bfloat16), We2_ref[0],
                              preferred_element_type=jnp.float32) + be2_ref[0]


def _stage_c(et, xs, We1, be1, We2, be2):
    wix = lambda i, et_ref: (jnp.minimum(et_ref[i], E - 1), 0, 0)
    grid_spec = pltpu.PrefetchScalarGridSpec(
        num_scalar_prefetch=1,
        grid=(NT,),
        in_specs=[
            pl.BlockSpec((BE, HID), lambda i, et_ref: (i, 0)),
            pl.BlockSpec((1, HID, MOEH), wix),
            pl.BlockSpec((1, 1, MOEH), wix),
            pl.BlockSpec((1, MOEH, HID), wix),
            pl.BlockSpec((1, 1, HID), wix),
        ],
        out_specs=pl.BlockSpec((BE, HID), lambda i, et_ref: (i, 0)),
    )
    return pl.pallas_call(
        _stage_c_body,
        grid_spec=grid_spec,
        out_shape=jax.ShapeDtypeStruct((NPAD, HID), jnp.float32),
        interpret=_INTERPRET,
    )(et, xs, We1.astype(jnp.bfloat16), be1.reshape(E, 1, MOEH),
      We2.astype(jnp.bfloat16), be2.reshape(E, 1, HID))


def _stage_d_body(gy0_ref, gy1_ref, tv0_ref, tv1_ref, Wp2_ref, bp2_ref, std_ref,
                  mu_ref, stdt_ref):
    y = tv0_ref[...] * gy0_ref[...] + tv1_ref[...] * gy1_ref[...]
    yr = jnp.maximum(y, 0.0)
    mu_ref[...] = jnp.tanh(jnp.dot(yr, Wp2_ref[...], preferred_element_type=jnp.float32) + bp2_ref[...])
    stdt_ref[...] = jnp.broadcast_to(std_ref[...], (BT, ACT))


def _stage_d(gy, tv0, tv1, Wp2, bp2, std):
    return pl.pallas_call(
        _stage_d_body,
        grid=(NTT,),
        in_specs=[
            pl.BlockSpec((BT, HID), lambda i: (i, 0)),
            pl.BlockSpec((BT, HID), lambda i: (i + NTT, 0)),
            pl.BlockSpec((BT, 1), lambda i: (i, 0)),
            pl.BlockSpec((BT, 1), lambda i: (i, 0)),
            pl.BlockSpec((HID, ACT), lambda i: (0, 0)),
            pl.BlockSpec((1, ACT), lambda i: (0, 0)),
            pl.BlockSpec((1, 1), lambda i: (0, 0)),
        ],
        out_specs=[
            pl.BlockSpec((BT, ACT), lambda i: (i, 0)),
            pl.BlockSpec((BT, ACT), lambda i: (i, 0)),
        ],
        out_shape=[
            jax.ShapeDtypeStruct((T, ACT), jnp.float32),
            jax.ShapeDtypeStruct((T, ACT), jnp.float32),
        ],
        interpret=_INTERPRET,
    )(gy, gy, tv0, tv1, Wp2, bp2.reshape(1, ACT), std.reshape(1, 1))


def kernel(obs, std, Wt, bt, ln_g, ln_b, Wp1, bp1, Wg1, bg1, Wg2, bg2,
           We1, be1, We2, be2, Wp2, bp2):
    x, tv0, tv1, p, et, aux = _stage_a(
        obs, Wt, bt, ln_g, ln_b, Wp1, bp1, Wg1, bg1, Wg2, bg2)
    xs = _sc_dispatch(x, p.reshape(NA))
    ys = _stage_c(et.reshape(NT), xs, We1, be1, We2, be2)
    gy = _sc_combine(ys, p.reshape(NA))
    mu, std_t = _stage_d(gy, tv0, tv1, Wp2, bp2, std)
    return (mu, std_t, aux[0, 0])


# speedup vs baseline: 1.2440x; 1.2440x over previous
import functools

import jax
import jax.numpy as jnp
from jax import lax
from jax.experimental import pallas as pl
from jax.experimental.pallas import tpu as pltpu
from jax.experimental.pallas import tpu_sc as plsc

T, REPR, FEAT, HID, GATE, E, MOEH, ACT = 2048, 2048, 1024, 1024, 256, 16, 512, 12
BT = 128
NTT = T // BT
BTA = 256
NTA = T // BTA
BE = 128
NT = 48
NPAD = NT * BE
NA = 2 * T

_INTERPRET = False

_NC, _NS = 2, 16
_NW = _NC * _NS
_CHUNK = 64


def _stage_a_body(obs_ref, Wt_ref, bt_ref, lng_ref, lnb_ref, Wp1_ref, bp1_ref,
                  Wg1_ref, bg1_ref, Wg2_ref, bg2_ref,
                  x_ref, tv0_ref, tv1_ref, p_ref, et_ref, dt_ref, aux_ref,
                  carry0, carry1, sp_acc, ls_acc, i0_s, i1_s, r0_s, r1_s):
    i = pl.program_id(0)
    h = jnp.dot(obs_ref[...], Wt_ref[...], preferred_element_type=jnp.float32) + bt_ref[...]
    mu_ = jnp.mean(h, axis=-1, keepdims=True)
    var = jnp.mean((h - mu_) ** 2, axis=-1, keepdims=True)
    h = (h - mu_) * jax.lax.rsqrt(var + 1e-5) * lng_ref[...] + lnb_ref[...]
    h = jnp.tanh(h)

    x = jnp.maximum(jnp.dot(h, Wp1_ref[...], preferred_element_type=jnp.float32) + bp1_ref[...], 0.0)
    x_ref[...] = x

    gh = jnp.maximum(jnp.dot(x, Wg1_ref[...], preferred_element_type=jnp.float32) + bg1_ref[...], 0.0)
    logits = jnp.dot(gh, Wg2_ref[...], preferred_element_type=jnp.float32) + bg2_ref[...]

    m = jnp.max(logits, axis=-1, keepdims=True)
    ex = jnp.exp(logits - m)
    probs = ex / jnp.sum(ex, axis=-1, keepdims=True)

    iot = jax.lax.broadcasted_iota(jnp.int32, probs.shape, 1)
    m0 = jnp.max(probs, axis=-1, keepdims=True)
    i0 = jnp.argmax(probs, axis=-1).reshape(BTA, 1)
    masked = jnp.where(iot == i0, -1.0, probs)
    m1 = jnp.max(masked, axis=-1, keepdims=True)
    i1 = jnp.argmax(masked, axis=-1).reshape(BTA, 1)
    s = m0 + m1 + 1e-9
    tv0 = m0 / s
    tv1 = m1 / s
    tv0_ref[...] = tv0
    tv1_ref[...] = tv1

    g0 = (iot == i0).astype(jnp.float32)
    g1 = (iot == i1).astype(jnp.float32)
    ir = jax.lax.broadcasted_iota(jnp.int32, (BTA, BTA), 0)
    ic = jax.lax.broadcasted_iota(jnp.int32, (BTA, BTA), 1)
    st = (ic < ir).astype(jnp.float32)

    @pl.when(i == 0)
    def _():
        carry0[...] = jnp.zeros_like(carry0)
        carry1[...] = jnp.zeros_like(carry1)
        sp_acc[...] = jnp.zeros_like(sp_acc)
        ls_acc[...] = jnp.zeros_like(ls_acc)

    cum0 = jnp.dot(st, g0, preferred_element_type=jnp.float32) + carry0[...]
    cum1 = jnp.dot(st, g1, preferred_element_type=jnp.float32) + carry1[...]
    rows = pl.ds(i * BTA, BTA)
    i0_s[rows, :] = i0
    i1_s[rows, :] = i1
    r0_s[rows, :] = jnp.sum(g0 * cum0, axis=1, keepdims=True)
    r1_s[rows, :] = jnp.sum(g1 * cum1, axis=1, keepdims=True)
    carry0[...] += jnp.sum(g0, axis=0, keepdims=True)
    carry1[...] += jnp.sum(g1, axis=0, keepdims=True)

    sp_acc[...] += jnp.sum(probs, axis=0, keepdims=True)
    ls_acc[...] += jnp.sum(g0 * (tv0 > 0.0).astype(jnp.float32)
                           + g1 * (tv1 > 0.0).astype(jnp.float32), axis=0, keepdims=True)

    @pl.when(i == NTA - 1)
    def _():
        c0 = carry0[...]
        c1 = carry1[...]
        pc = jnp.ceil((c0 + c1) / BE) * BE
        fr = jax.lax.broadcasted_iota(jnp.int32, (E, E), 0)
        fc = jax.lax.broadcasted_iota(jnp.int32, (E, E), 1)
        mstrict = (fr < fc).astype(jnp.float32)
        offp = jnp.dot(pc, mstrict, preferred_element_type=jnp.float32)
        cum_end = offp + pc

        iota_e = jax.lax.broadcasted_iota(jnp.int32, (T, E), 1)
        oh0 = (iota_e == i0_s[...]).astype(jnp.float32)
        oh1 = (iota_e == i1_s[...]).astype(jnp.float32)
        p0 = jnp.sum(oh0 * offp, axis=1, keepdims=True) + r0_s[...]
        p1 = jnp.sum(oh1 * (offp + c0), axis=1, keepdims=True) + r1_s[...]
        p_ref[0:T, :] = p0.astype(jnp.int32)
        p_ref[T:NA, :] = p1.astype(jnp.int32)

        itn = jax.lax.broadcasted_iota(jnp.int32, (NT, E), 0).astype(jnp.float32)
        it = itn * BE
        et = jnp.sum((cum_end <= it).astype(jnp.float32), axis=1, keepdims=True)
        et_ref[...] = et.astype(jnp.int32)
        nact1 = jnp.sum(pc) / BE - 1.0
        dt_ref[...] = jnp.minimum(itn, nact1)[:, 0:1].astype(jnp.int32)

        aux_ref[...] = ((E / (T * float(T))) * jnp.sum(sp_acc[...] * ls_acc[...])).reshape(1, 1)


def _stage_a(obs, Wt, bt, ln_g, ln_b, Wp1, bp1, Wg1, bg1, Wg2, bg2):
    row = lambda v: v.reshape(1, -1)
    const = lambda i: (0, 0)
    tok = lambda i: (i, 0)
    return pl.pallas_call(
        _stage_a_body,
        grid=(NTA,),
        in_specs=[
            pl.BlockSpec((BTA, REPR), tok),
            pl.BlockSpec((REPR, FEAT), const),
            pl.BlockSpec((1, FEAT), const),
            pl.BlockSpec((1, FEAT), const),
            pl.BlockSpec((1, FEAT), const),
            pl.BlockSpec((FEAT, HID), const),
            pl.BlockSpec((1, HID), const),
            pl.BlockSpec((HID, GATE), const),
            pl.BlockSpec((1, GATE), const),
            pl.BlockSpec((GATE, E), const),
            pl.BlockSpec((1, E), const),
        ],
        out_specs=[
            pl.BlockSpec((BTA, HID), tok),
            pl.BlockSpec((BTA, 1), tok),
            pl.BlockSpec((BTA, 1), tok),
            pl.BlockSpec((NA, 1), const),
            pl.BlockSpec((NT, 1), const),
            pl.BlockSpec((NT, 1), const),
            pl.BlockSpec((1, 1), const),
        ],
        out_shape=[
            jax.ShapeDtypeStruct((T, HID), jnp.float32),
            jax.ShapeDtypeStruct((T, 1), jnp.float32),
            jax.ShapeDtypeStruct((T, 1), jnp.float32),
            jax.ShapeDtypeStruct((NA, 1), jnp.int32),
            jax.ShapeDtypeStruct((NT, 1), jnp.int32),
            jax.ShapeDtypeStruct((NT, 1), jnp.int32),
            jax.ShapeDtypeStruct((1, 1), jnp.float32),
        ],
        scratch_shapes=[pltpu.VMEM((1, E), jnp.float32)] * 4
        + [pltpu.VMEM((T, 1), jnp.int32)] * 2
        + [pltpu.VMEM((T, 1), jnp.float32)] * 2,
        interpret=_INTERPRET,
    )(obs, Wt, row(bt), row(ln_g), row(ln_b), Wp1, row(bp1), Wg1, row(bg1), Wg2, row(bg2))


def _sc_dispatch(x, p):
    @functools.partial(
        pl.kernel,
        mesh=plsc.VectorSubcoreMesh(core_axis_name="c", subcore_axis_name="s"),
        out_type=jax.ShapeDtypeStruct((NPAD, HID), jnp.float32),
        scratch_types=[
            pltpu.VMEM((_CHUNK,), jnp.int32),
            pltpu.VMEM((_CHUNK, HID), jnp.float32),
            pltpu.SemaphoreType.DMA,
        ],
    )
    def k(x_hbm, p_hbm, xs_hbm, idx_v, rows_v, sem):
        wid = lax.axis_index("s") * _NC + lax.axis_index("c")
        per_w = NA // _NW
        for c in range(per_w // _CHUNK):
            base = wid * per_w + c * _CHUNK
            tbase = jnp.where(base >= T, base - T, base)
            pltpu.sync_copy(p_hbm.at[pl.ds(base, _CHUNK)], idx_v)
            pltpu.sync_copy(x_hbm.at[pl.ds(tbase, _CHUNK)], rows_v)
            pltpu.async_copy(rows_v, xs_hbm.at[idx_v], sem).wait()

    return k(x, p)


def _sc_combine(ys, p):
    @functools.partial(
        pl.kernel,
        mesh=plsc.VectorSubcoreMesh(core_axis_name="c", subcore_axis_name="s"),
        out_type=jax.ShapeDtypeStruct((NA, HID), jnp.float32),
        scratch_types=[
            pltpu.VMEM((_CHUNK,), jnp.int32),
            pltpu.VMEM((_CHUNK, HID), jnp.float32),
            pltpu.SemaphoreType.DMA,
        ],
    )
    def k(ys_hbm, p_hbm, gy_hbm, idx_v, rows_v, sem):
        wid = lax.axis_index("s") * _NC + lax.axis_index("c")
        per_w = NA // _NW
        for c in range(per_w // _CHUNK):
            base = wid * per_w + c * _CHUNK
            pltpu.sync_copy(p_hbm.at[pl.ds(base, _CHUNK)], idx_v)
            pltpu.async_copy(ys_hbm.at[idx_v], rows_v, sem).wait()
            pltpu.sync_copy(rows_v, gy_hbm.at[pl.ds(base, _CHUNK)])

    return k(ys, p)


def _stage_c_body(et_sref, dt_sref, xs_ref, We1_ref, be1_ref, We2_ref, be2_ref, ys_ref):
    etv = et_sref[pl.program_id(0)]

    @pl.when(etv < E)
    def _():
        eh = jnp.maximum(jnp.dot(xs_ref[...], We1_ref[0], preferred_element_type=jnp.float32)
                         + be1_ref[0], 0.0)
        ys_ref[...] = jnp.dot(eh, We2_ref[0], preferred_element_type=jnp.float32) + be2_ref[0]


def _stage_c(et, dt, xs, We1, be1, We2, be2):
    wix = lambda i, et_ref, dt_ref: (jnp.minimum(et_ref[i], E - 1), 0, 0)
    dix = lambda i, et_ref, dt_ref: (dt_ref[i], 0)
    grid_spec = pltpu.PrefetchScalarGridSpec(
        num_scalar_prefetch=2,
        grid=(NT,),
        in_specs=[
            pl.BlockSpec((BE, HID), dix),
            pl.BlockSpec((1, HID, MOEH), wix),
            pl.BlockSpec((1, 1, MOEH), wix),
            pl.BlockSpec((1, MOEH, HID), wix),
            pl.BlockSpec((1, 1, HID), wix),
        ],
        out_specs=pl.BlockSpec((BE, HID), dix),
    )
    return pl.pallas_call(
        _stage_c_body,
        grid_spec=grid_spec,
        out_shape=jax.ShapeDtypeStruct((NPAD, HID), jnp.float32),
        interpret=_INTERPRET,
    )(et, dt, xs, We1, be1.reshape(E, 1, MOEH), We2, be2.reshape(E, 1, HID))


def _stage_d_body(gy0_ref, gy1_ref, tv0_ref, tv1_ref, Wp2_ref, bp2_ref, std_ref,
                  mu_ref, stdt_ref):
    y = tv0_ref[...] * gy0_ref[...] + tv1_ref[...] * gy1_ref[...]
    yr = jnp.maximum(y, 0.0)
    mu_ref[...] = jnp.tanh(jnp.dot(yr, Wp2_ref[...], preferred_element_type=jnp.float32) + bp2_ref[...])
    stdt_ref[...] = jnp.broadcast_to(std_ref[...], (BT, ACT))


def _stage_d(gy, tv0, tv1, Wp2, bp2, std):
    return pl.pallas_call(
        _stage_d_body,
        grid=(NTT,),
        in_specs=[
            pl.BlockSpec((BT, HID), lambda i: (i, 0)),
            pl.BlockSpec((BT, HID), lambda i: (i + NTT, 0)),
            pl.BlockSpec((BT, 1), lambda i: (i, 0)),
            pl.BlockSpec((BT, 1), lambda i: (i, 0)),
            pl.BlockSpec((HID, ACT), lambda i: (0, 0)),
            pl.BlockSpec((1, ACT), lambda i: (0, 0)),
            pl.BlockSpec((1, 1), lambda i: (0, 0)),
        ],
        out_specs=[
            pl.BlockSpec((BT, ACT), lambda i: (i, 0)),
            pl.BlockSpec((BT, ACT), lambda i: (i, 0)),
        ],
        out_shape=[
            jax.ShapeDtypeStruct((T, ACT), jnp.float32),
            jax.ShapeDtypeStruct((T, ACT), jnp.float32),
        ],
        interpret=_INTERPRET,
    )(gy, gy, tv0, tv1, Wp2, bp2.reshape(1, ACT), std.reshape(1, 1))


def kernel(obs, std, Wt, bt, ln_g, ln_b, Wp1, bp1, Wg1, bg1, Wg2, bg2,
           We1, be1, We2, be2, Wp2, bp2):
    x, tv0, tv1, p, et, dt, aux = _stage_a(
        obs, Wt, bt, ln_g, ln_b, Wp1, bp1, Wg1, bg1, Wg2, bg2)
    xs = _sc_dispatch(x, p.reshape(NA))
    ys = _stage_c(et.reshape(NT), dt.reshape(NT), xs, We1, be1, We2, be2)
    gy = _sc_combine(ys, p.reshape(NA))
    mu, std_t = _stage_d(gy, tv0, tv1, Wp2, bp2, std)
    return (mu, std_t, aux[0, 0])


# speedup vs baseline: 1.2666x; 1.0181x over previous
import functools

import jax
import jax.numpy as jnp
from jax import lax
from jax.experimental import pallas as pl
from jax.experimental.pallas import tpu as pltpu
from jax.experimental.pallas import tpu_sc as plsc

T, REPR, FEAT, HID, GATE, E, MOEH, ACT = 2048, 2048, 1024, 1024, 256, 16, 512, 12
BT = 128
NTT = T // BT
BTA = 512
NTA = T // BTA
BE = 128
NT = 48
NPAD = NT * BE
NA = 2 * T

_INTERPRET = False

_NC, _NS = 2, 16
_NW = _NC * _NS
_CHUNK = 64


def _stage_a_body(obs_ref, Wt_ref, bt_ref, lng_ref, lnb_ref, Wp1_ref, bp1_ref,
                  Wg1_ref, bg1_ref, Wg2_ref, bg2_ref,
                  x_ref, tv0_ref, tv1_ref, p_ref, et_ref, dt_ref, aux_ref,
                  carry0, carry1, sp_acc, ls_acc, i0_s, i1_s, r0_s, r1_s):
    i = pl.program_id(0)
    h = jnp.dot(obs_ref[...], Wt_ref[...], preferred_element_type=jnp.float32) + bt_ref[...]
    mu_ = jnp.mean(h, axis=-1, keepdims=True)
    var = jnp.mean((h - mu_) ** 2, axis=-1, keepdims=True)
    h = (h - mu_) * jax.lax.rsqrt(var + 1e-5) * lng_ref[...] + lnb_ref[...]
    h = jnp.tanh(h)

    x = jnp.maximum(jnp.dot(h, Wp1_ref[...], preferred_element_type=jnp.float32) + bp1_ref[...], 0.0)
    x_ref[...] = x

    gh = jnp.maximum(jnp.dot(x, Wg1_ref[...], preferred_element_type=jnp.float32) + bg1_ref[...], 0.0)
    logits = jnp.dot(gh, Wg2_ref[...], preferred_element_type=jnp.float32) + bg2_ref[...]

    m = jnp.max(logits, axis=-1, keepdims=True)
    ex = jnp.exp(logits - m)
    probs = ex / jnp.sum(ex, axis=-1, keepdims=True)

    iot = jax.lax.broadcasted_iota(jnp.int32, probs.shape, 1)
    m0 = jnp.max(probs, axis=-1, keepdims=True)
    i0 = jnp.argmax(probs, axis=-1).reshape(BTA, 1)
    masked = jnp.where(iot == i0, -1.0, probs)
    m1 = jnp.max(masked, axis=-1, keepdims=True)
    i1 = jnp.argmax(masked, axis=-1).reshape(BTA, 1)
    s = m0 + m1 + 1e-9
    tv0 = m0 / s
    tv1 = m1 / s
    tv0_ref[...] = tv0
    tv1_ref[...] = tv1

    g0 = (iot == i0).astype(jnp.float32)
    g1 = (iot == i1).astype(jnp.float32)
    ir = jax.lax.broadcasted_iota(jnp.int32, (BTA, BTA), 0)
    ic = jax.lax.broadcasted_iota(jnp.int32, (BTA, BTA), 1)
    st = (ic < ir).astype(jnp.float32)

    @pl.when(i == 0)
    def _():
        carry0[...] = jnp.zeros_like(carry0)
        carry1[...] = jnp.zeros_like(carry1)
        sp_acc[...] = jnp.zeros_like(sp_acc)
        ls_acc[...] = jnp.zeros_like(ls_acc)

    cum0 = jnp.dot(st, g0, preferred_element_type=jnp.float32) + carry0[...]
    cum1 = jnp.dot(st, g1, preferred_element_type=jnp.float32) + carry1[...]
    rows = pl.ds(i * BTA, BTA)
    i0_s[rows, :] = i0
    i1_s[rows, :] = i1
    r0_s[rows, :] = jnp.sum(g0 * cum0, axis=1, keepdims=True)
    r1_s[rows, :] = jnp.sum(g1 * cum1, axis=1, keepdims=True)
    carry0[...] += jnp.sum(g0, axis=0, keepdims=True)
    carry1[...] += jnp.sum(g1, axis=0, keepdims=True)

    sp_acc[...] += jnp.sum(probs, axis=0, keepdims=True)
    ls_acc[...] += jnp.sum(g0 * (tv0 > 0.0).astype(jnp.float32)
                           + g1 * (tv1 > 0.0).astype(jnp.float32), axis=0, keepdims=True)

    @pl.when(i == NTA - 1)
    def _():
        c0 = carry0[...]
        c1 = carry1[...]
        pc = jnp.ceil((c0 + c1) / BE) * BE
        fr = jax.lax.broadcasted_iota(jnp.int32, (E, E), 0)
        fc = jax.lax.broadcasted_iota(jnp.int32, (E, E), 1)
        mstrict = (fr < fc).astype(jnp.float32)
        offp = jnp.dot(pc, mstrict, preferred_element_type=jnp.float32)
        cum_end = offp + pc

        iota_e = jax.lax.broadcasted_iota(jnp.int32, (T, E), 1)
        oh0 = (iota_e == i0_s[...]).astype(jnp.float32)
        oh1 = (iota_e == i1_s[...]).astype(jnp.float32)
        p0 = jnp.sum(oh0 * offp, axis=1, keepdims=True) + r0_s[...]
        p1 = jnp.sum(oh1 * (offp + c0), axis=1, keepdims=True) + r1_s[...]
        p_ref[0:T, :] = p0.astype(jnp.int32)
        p_ref[T:NA, :] = p1.astype(jnp.int32)

        itn = jax.lax.broadcasted_iota(jnp.int32, (NT, E), 0).astype(jnp.float32)
        it = itn * BE
        et = jnp.sum((cum_end <= it).astype(jnp.float32), axis=1, keepdims=True)
        et_ref[...] = et.astype(jnp.int32)
        nact1 = jnp.sum(pc) / BE - 1.0
        dt_ref[...] = jnp.minimum(itn, nact1)[:, 0:1].astype(jnp.int32)

        aux_ref[...] = ((E / (T * float(T))) * jnp.sum(sp_acc[...] * ls_acc[...])).reshape(1, 1)


def _stage_a(obs, Wt, bt, ln_g, ln_b, Wp1, bp1, Wg1, bg1, Wg2, bg2):
    row = lambda v: v.reshape(1, -1)
    const = lambda i: (0, 0)
    tok = lambda i: (i, 0)
    return pl.pallas_call(
        _stage_a_body,
        grid=(NTA,),
        in_specs=[
            pl.BlockSpec((BTA, REPR), tok),
            pl.BlockSpec((REPR, FEAT), const),
            pl.BlockSpec((1, FEAT), const),
            pl.BlockSpec((1, FEAT), const),
            pl.BlockSpec((1, FEAT), const),
            pl.BlockSpec((FEAT, HID), const),
            pl.BlockSpec((1, HID), const),
            pl.BlockSpec((HID, GATE), const),
            pl.BlockSpec((1, GATE), const),
            pl.BlockSpec((GATE, E), const),
            pl.BlockSpec((1, E), const),
        ],
        out_specs=[
            pl.BlockSpec((BTA, HID), tok),
            pl.BlockSpec((BTA, 1), tok),
            pl.BlockSpec((BTA, 1), tok),
            pl.BlockSpec((NA, 1), const),
            pl.BlockSpec((NT, 1), const),
            pl.BlockSpec((NT, 1), const),
            pl.BlockSpec((1, 1), const),
        ],
        out_shape=[
            jax.ShapeDtypeStruct((T, HID), jnp.float32),
            jax.ShapeDtypeStruct((T, 1), jnp.float32),
            jax.ShapeDtypeStruct((T, 1), jnp.float32),
            jax.ShapeDtypeStruct((NA, 1), jnp.int32),
            jax.ShapeDtypeStruct((NT, 1), jnp.int32),
            jax.ShapeDtypeStruct((NT, 1), jnp.int32),
            jax.ShapeDtypeStruct((1, 1), jnp.float32),
        ],
        scratch_shapes=[pltpu.VMEM((1, E), jnp.float32)] * 4
        + [pltpu.VMEM((T, 1), jnp.int32)] * 2
        + [pltpu.VMEM((T, 1), jnp.float32)] * 2,
        interpret=_INTERPRET,
    )(obs, Wt, row(bt), row(ln_g), row(ln_b), Wp1, row(bp1), Wg1, row(bg1), Wg2, row(bg2))


def _sc_dispatch(x, p):
    @functools.partial(
        pl.kernel,
        mesh=plsc.VectorSubcoreMesh(core_axis_name="c", subcore_axis_name="s"),
        out_type=jax.ShapeDtypeStruct((NPAD, HID), jnp.float32),
        scratch_types=[
            pltpu.VMEM((_CHUNK,), jnp.int32),
            pltpu.VMEM((_CHUNK, HID), jnp.float32),
            pltpu.SemaphoreType.DMA,
        ],
    )
    def k(x_hbm, p_hbm, xs_hbm, idx_v, rows_v, sem):
        wid = lax.axis_index("s") * _NC + lax.axis_index("c")
        per_w = NA // _NW
        for c in range(per_w // _CHUNK):
            base = wid * per_w + c * _CHUNK
            tbase = jnp.where(base >= T, base - T, base)
            pltpu.sync_copy(p_hbm.at[pl.ds(base, _CHUNK)], idx_v)
            pltpu.sync_copy(x_hbm.at[pl.ds(tbase, _CHUNK)], rows_v)
            pltpu.async_copy(rows_v, xs_hbm.at[idx_v], sem).wait()

    return k(x, p)


def _sc_combine(ys, p):
    @functools.partial(
        pl.kernel,
        mesh=plsc.VectorSubcoreMesh(core_axis_name="c", subcore_axis_name="s"),
        out_type=jax.ShapeDtypeStruct((NA, HID), jnp.float32),
        scratch_types=[
            pltpu.VMEM((_CHUNK,), jnp.int32),
            pltpu.VMEM((_CHUNK, HID), jnp.float32),
            pltpu.SemaphoreType.DMA,
        ],
    )
    def k(ys_hbm, p_hbm, gy_hbm, idx_v, rows_v, sem):
        wid = lax.axis_index("s") * _NC + lax.axis_index("c")
        per_w = NA // _NW
        for c in range(per_w // _CHUNK):
            base = wid * per_w + c * _CHUNK
            pltpu.sync_copy(p_hbm.at[pl.ds(base, _CHUNK)], idx_v)
            pltpu.async_copy(ys_hbm.at[idx_v], rows_v, sem).wait()
            pltpu.sync_copy(rows_v, gy_hbm.at[pl.ds(base, _CHUNK)])

    return k(ys, p)


def _stage_c_body(et_sref, dt_sref, xs_ref, We1_ref, be1_ref, We2_ref, be2_ref, ys_ref):
    etv = et_sref[pl.program_id(0)]

    @pl.when(etv < E)
    def _():
        eh = jnp.maximum(jnp.dot(xs_ref[...], We1_ref[0], preferred_element_type=jnp.float32)
                         + be1_ref[0], 0.0)
        ys_ref[...] = jnp.dot(eh, We2_ref[0], preferred_element_type=jnp.float32) + be2_ref[0]


def _stage_c(et, dt, xs, We1, be1, We2, be2):
    wix = lambda i, et_ref, dt_ref: (jnp.minimum(et_ref[i], E - 1), 0, 0)
    dix = lambda i, et_ref, dt_ref: (dt_ref[i], 0)
    grid_spec = pltpu.PrefetchScalarGridSpec(
        num_scalar_prefetch=2,
        grid=(NT,),
        in_specs=[
            pl.BlockSpec((BE, HID), dix),
            pl.BlockSpec((1, HID, MOEH), wix),
            pl.BlockSpec((1, 1, MOEH), wix),
            pl.BlockSpec((1, MOEH, HID), wix),
            pl.BlockSpec((1, 1, HID), wix),
        ],
        out_specs=pl.BlockSpec((BE, HID), dix),
    )
    return pl.pallas_call(
        _stage_c_body,
        grid_spec=grid_spec,
        out_shape=jax.ShapeDtypeStruct((NPAD, HID), jnp.float32),
        interpret=_INTERPRET,
    )(et, dt, xs, We1, be1.reshape(E, 1, MOEH), We2, be2.reshape(E, 1, HID))


def _stage_d_body(gy_ref, tv0_ref, tv1_ref, Wp2_ref, bp2_ref, std_ref,
                  mu_ref, stdt_ref):
    y = tv0_ref[...] * gy_ref[0] + tv1_ref[...] * gy_ref[1]
    yr = jnp.maximum(y, 0.0)
    mu_ref[...] = jnp.tanh(jnp.dot(yr, Wp2_ref[...], preferred_element_type=jnp.float32) + bp2_ref[...])
    stdt_ref[...] = jnp.broadcast_to(std_ref[...], (BT, ACT))


def _stage_d(gy, tv0, tv1, Wp2, bp2, std):
    return pl.pallas_call(
        _stage_d_body,
        grid=(NTT,),
        in_specs=[
            pl.BlockSpec((2, BT, HID), lambda i: (0, i, 0)),
            pl.BlockSpec((BT, 1), lambda i: (i, 0)),
            pl.BlockSpec((BT, 1), lambda i: (i, 0)),
            pl.BlockSpec((HID, ACT), lambda i: (0, 0)),
            pl.BlockSpec((1, ACT), lambda i: (0, 0)),
            pl.BlockSpec((1, 1), lambda i: (0, 0)),
        ],
        out_specs=[
            pl.BlockSpec((BT, ACT), lambda i: (i, 0)),
            pl.BlockSpec((BT, ACT), lambda i: (i, 0)),
        ],
        out_shape=[
            jax.ShapeDtypeStruct((T, ACT), jnp.float32),
            jax.ShapeDtypeStruct((T, ACT), jnp.float32),
        ],
        interpret=_INTERPRET,
    )(gy.reshape(2, T, HID), tv0, tv1, Wp2, bp2.reshape(1, ACT), std.reshape(1, 1))


def kernel(obs, std, Wt, bt, ln_g, ln_b, Wp1, bp1, Wg1, bg1, Wg2, bg2,
           We1, be1, We2, be2, Wp2, bp2):
    x, tv0, tv1, p, et, dt, aux = _stage_a(
        obs, Wt, bt, ln_g, ln_b, Wp1, bp1, Wg1, bg1, Wg2, bg2)
    xs = _sc_dispatch(x, p.reshape(NA))
    ys = _stage_c(et.reshape(NT), dt.reshape(NT), xs, We1, be1, We2, be2)
    gy = _sc_combine(ys, p.reshape(NA))
    mu, std_t = _stage_d(gy, tv0, tv1, Wp2, bp2, std)
    return (mu, std_t, aux[0, 0])
